# Initial kernel scaffold; baseline (speedup 1.0000x reference)
import jax
import jax.numpy as jnp
from jax.experimental import pallas as pl


def kernel(indices, TT_core):
    raise NotImplementedError("write your pallas kernel here")


# speedup vs baseline: 2.6296x; 2.6296x over previous
import functools

import jax
import jax.numpy as jnp
from jax import lax
from jax.experimental import pallas as pl
from jax.experimental.pallas import tpu as pltpu
from jax.experimental.pallas import tpu_sc as plsc

R1, R2 = 16, 16
N1, N2 = 256, 256
B = 16384

NC, NS, L = 2, 16, 16
NW = NC * NS
BW = B // NW
ROWS_W = BW * R1
IDX_MINOR = 128
NSTREAM = ROWS_W // IDX_MINOR
GROUP_ROWS = 2048
NGROUP = ROWS_W // GROUP_ROWS
SPG = GROUP_ROWS // IDX_MINOR


def _build():
    mesh = plsc.VectorSubcoreMesh(
        core_axis_name="c", subcore_axis_name="s",
        num_cores=NC, num_subcores=NS)

    @functools.partial(
        pl.kernel,
        out_type=jax.ShapeDtypeStruct((B * R1, R2), jnp.float32),
        mesh=mesh,
        compiler_params=pltpu.CompilerParams(
            needs_layout_passes=False, use_tc_tiling_on_sc=False),
        scratch_types=[
            pltpu.VMEM((BW * 2,), jnp.int32),
            pltpu.VMEM((NSTREAM, IDX_MINOR), jnp.int32),
            pltpu.VMEM((GROUP_ROWS, R2), jnp.float32),
            pltpu.VMEM((GROUP_ROWS, R2), jnp.float32),
            pltpu.SemaphoreType.DMA,
            pltpu.SemaphoreType.DMA,
            pltpu.SemaphoreType.DMA,
        ],
    )
    def run(idx_hbm, table_hbm, out_hbm,
            pair_v, idx2_v, rows0, rows1, gsem0, gsem1, osem):
        w = lax.axis_index("s") * NC + lax.axis_index("c")
        b0 = w * BW
        pltpu.sync_copy(idx_hbm.at[pl.ds(b0 * 2, BW * 2)], pair_v)

        iota = lax.iota(jnp.int32, L)
        off = iota * (N1 * N2)
        vpr = IDX_MINOR // L

        def expand(j, carry):
            jb = jnp.full((L,), 2 * j, jnp.int32)
            i0 = plsc.load_gather(pair_v, [jb])
            i1 = plsc.load_gather(pair_v, [jb + 1])
            vec = i0 * N2 + i1 + off
            rows = jnp.full((L,), j // vpr, jnp.int32)
            cols = (j % vpr) * L + iota
            plsc.store_scatter(idx2_v, [rows, cols], vec)
            return carry

        lax.fori_loop(0, BW, expand, 0)

        bufs = (rows0, rows1)
        gsems = (gsem0, gsem1)

        def fire(g):
            buf, sem = bufs[g % 2], gsems[g % 2]
            return [
                pltpu.async_copy(
                    table_hbm.at[idx2_v.at[g * SPG + i]],
                    buf.at[pl.ds(i * IDX_MINOR, IDX_MINOR)],
                    sem)
                for i in range(SPG)
            ]

        pending = fire(0)
        ocps = [None, None]
        for g in range(NGROUP):
            for c in pending:
                c.wait()
            if g + 1 < NGROUP:
                if ocps[(g + 1) % 2] is not None:
                    ocps[(g + 1) % 2].wait()
                pending = fire(g + 1)
            ocps[g % 2] = pltpu.async_copy(
                bufs[g % 2],
                out_hbm.at[pl.ds(b0 * R1 + g * GROUP_ROWS, GROUP_ROWS)],
                osem)
        for o in ocps:
            if o is not None:
                o.wait()

    return run


_tt_gather = _build()


@jax.jit
def kernel(indices, TT_core):
    table = TT_core.reshape(R1 * N1 * N2, R2)
    out = _tt_gather(indices.reshape(B * 2), table)
    return out.reshape(B, R1, R2)


# speedup vs baseline: 8.5614x; 3.2558x over previous
import functools

import jax
import jax.numpy as jnp
from jax import lax
from jax.experimental import pallas as pl
from jax.experimental.pallas import tpu as pltpu
from jax.experimental.pallas import tpu_sc as plsc

R1, R2 = 16, 16
N1, N2 = 256, 256
B = 16384

NC, NS, L = 2, 16, 16
NW = NC * NS
BW = B // NW
CB = BW // 128
NSTREAM = 64
GE = NSTREAM * 128


def _build():
    mesh = plsc.VectorSubcoreMesh(
        core_axis_name="c", subcore_axis_name="s",
        num_cores=NC, num_subcores=NS)

    @functools.partial(
        pl.kernel,
        out_type=jax.ShapeDtypeStruct((R1 * 2, B * 8), jnp.float32),
        mesh=mesh,
        compiler_params=pltpu.CompilerParams(
            needs_layout_passes=False, use_tc_tiling_on_sc=False),
        scratch_types=[
            pltpu.VMEM((CB, 2, 128), jnp.int32),
            pltpu.VMEM((BW,), jnp.int32),
            pltpu.VMEM((NSTREAM, 128), jnp.int32),
            pltpu.VMEM((NSTREAM, 128), jnp.int32),
            pltpu.VMEM((GE,), jnp.float32),
            pltpu.VMEM((GE,), jnp.float32),
            pltpu.SemaphoreType.DMA,
            pltpu.SemaphoreType.DMA,
            pltpu.SemaphoreType.DMA,
        ],
    )
    def run(idx_hbm, tab_hbm, out_hbm,
            pair_v, gb_v, id0, id1, dat0, dat1, gsem0, gsem1, osem):
        w = lax.axis_index("s") * NC + lax.axis_index("c")
        pltpu.sync_copy(idx_hbm.at[pl.ds(w * CB, CB)], pair_v)

        for c4 in range(CB):
            for ch in range(8):
                i0 = pair_v[c4, 0, pl.ds(ch * L, L)]
                i1 = pair_v[c4, 1, pl.ds(ch * L, L)]
                gb_v[pl.ds(c4 * 128 + ch * L, L)] = (
                    i0 * 4096 + (i1 >> 7) * 1024 + (i1 & 127))

        ids = (id0, id1)
        dats = (dat0, dat1)
        gsems = (gsem0, gsem1)

        def expand(r1, idv):
            def body(m, carry):
                c2 = r1 * 1048576 + (m // 32) * 2048 + (m % 8) * 128
                gb0 = ((m // 8) % 4) * 128
                for ch in range(8):
                    gb = gb_v[pl.ds(gb0 + ch * L, L)]
                    idv[m, pl.ds(ch * L, L)] = gb + c2
                return carry
            lax.fori_loop(0, NSTREAM, body, 0)

        def fire(r1):
            idv, datv, sem = ids[r1 % 2], dats[r1 % 2], gsems[r1 % 2]
            def body(s, carry):
                pltpu.async_copy(
                    tab_hbm.at[idv.at[s]], datv.at[pl.ds(s * 128, 128)], sem)
                return carry
            lax.fori_loop(0, NSTREAM, body, 0)

        def drain_gather(r1):
            pltpu.make_async_copy(
                tab_hbm.at[pl.ds(0, GE)], dats[r1 % 2], gsems[r1 % 2]).wait()

        def fire_out(r1):
            datv = dats[r1 % 2]
            half = GE // 2
            pltpu.async_copy(
                datv.at[pl.ds(0, half)],
                out_hbm.at[2 * r1, pl.ds(w * half, half)], osem)
            pltpu.async_copy(
                datv.at[pl.ds(half, half)],
                out_hbm.at[2 * r1 + 1, pl.ds(w * half, half)], osem)

        def drain_out(r1):
            pltpu.make_async_copy(
                tab_hbm.at[pl.ds(0, GE)], dats[r1 % 2], osem).wait()

        expand(0, ids[0])
        fire(0)
        for g in range(1, R1):
            expand(g, ids[g % 2])
            if g >= 2:
                drain_out(g)
            fire(g)
            drain_gather(g - 1)
            fire_out(g - 1)
        drain_out(R1)
        drain_gather(R1 - 1)
        fire_out(R1 - 1)
        drain_out(R1 + 1)

    return run


_tt_gather = _build()


@jax.jit
def kernel(indices, TT_core):
    idx3 = indices.reshape(128, 128, 2).transpose(0, 2, 1)
    tab = (TT_core.reshape(R1, N1, 2, 128, 2, 8)
           .transpose(0, 1, 4, 2, 5, 3).reshape(R1 * N1 * N2 * R2))
    out2 = _tt_gather(idx3, tab)
    return (out2.reshape(R1, 2, 128, 8, 128)
            .transpose(2, 4, 0, 1, 3).reshape(B, R1, R2))


# speedup vs baseline: 8.7030x; 1.0165x over previous
import functools

import jax
import jax.numpy as jnp
from jax import lax
from jax.experimental import pallas as pl
from jax.experimental.pallas import tpu as pltpu
from jax.experimental.pallas import tpu_sc as plsc

R1, R2 = 16, 16
N1, N2 = 256, 256
B = 16384

NC, NS, L = 2, 16, 16
NW = NC * NS
BW = B // NW
CB = BW // 128
NROW = 64
RSTRIDE = N1 * N2 * R2


def _build():
    mesh = plsc.VectorSubcoreMesh(
        core_axis_name="c", subcore_axis_name="s",
        num_cores=NC, num_subcores=NS)

    @functools.partial(
        pl.kernel,
        out_type=jax.ShapeDtypeStruct((R1 * 2, B * 8), jnp.float32),
        mesh=mesh,
        compiler_params=pltpu.CompilerParams(
            needs_layout_passes=False, use_tc_tiling_on_sc=False),
        scratch_types=[
            pltpu.VMEM((CB, 2, 128), jnp.int32),
            pltpu.VMEM((BW,), jnp.int32),
            pltpu.VMEM((NROW, 128), jnp.int32),
            pltpu.VMEM((NROW * 128,), jnp.float32),
            pltpu.VMEM((NROW * 128,), jnp.float32),
            pltpu.SemaphoreType.DMA,
            pltpu.SemaphoreType.DMA,
            pltpu.SemaphoreType.DMA,
        ],
    )
    def run(idx_hbm, tab_hbm, out_hbm,
            pair_v, gb_v, idq, dat0, dat1, gsem0, gsem1, osem):
        w = lax.axis_index("s") * NC + lax.axis_index("c")
        pltpu.sync_copy(idx_hbm.at[pl.ds(w * CB, CB)], pair_v)

        for c4 in range(CB):
            for ch in range(8):
                i0 = pair_v[c4, 0, pl.ds(ch * L, L)]
                i1 = pair_v[c4, 1, pl.ds(ch * L, L)]
                gb_v[pl.ds(c4 * 128 + ch * L, L)] = (
                    i0 * 4096 + (i1 >> 7) * 1024 + (i1 & 127))

        def expand(m, carry):
            c2 = (m // 32) * 2048 + (m % 8) * 128
            gb0 = ((m // 8) % 4) * 128
            for ch in range(8):
                gb = gb_v[pl.ds(gb0 + ch * L, L)]
                idq[m, pl.ds(ch * L, L)] = gb + c2
            return carry
        lax.fori_loop(0, NROW, expand, 0)

        dats = (dat0, dat1)
        gsems = (gsem0, gsem1)

        def fire(g):
            tslice = tab_hbm.at[pl.ds(g * RSTRIDE, RSTRIDE)]
            datv, sem = dats[g % 2], gsems[g % 2]
            def body(s, carry):
                pltpu.async_copy(
                    tslice.at[idq.at[s]], datv.at[pl.ds(s * 128, 128)], sem)
                return carry
            lax.fori_loop(0, NROW, body, 0)

        def drain(g, sem):
            pltpu.make_async_copy(
                tab_hbm.at[pl.ds(0, NROW * 128)], dats[g % 2], sem).wait()

        def fire_out(g):
            for t in range(2):
                pltpu.async_copy(
                    dats[g % 2].at[pl.ds(t * 4096, 4096)],
                    out_hbm.at[2 * g + t, pl.ds(w * 4096, 4096)], osem)

        fire(0)
        for g in range(1, R1):
            if g >= 2:
                drain(g, osem)
            fire(g)
            drain(g - 1, gsems[(g - 1) % 2])
            fire_out(g - 1)
        drain(R1, osem)
        drain(R1 - 1, gsems[(R1 - 1) % 2])
        fire_out(R1 - 1)
        drain(R1 + 1, osem)

    return run


_tt_gather = _build()


@jax.jit
def kernel(indices, TT_core):
    idx3 = indices.reshape(128, 128, 2).transpose(0, 2, 1)
    tab = (TT_core.reshape(R1, N1, 2, 128, 2, 8)
           .transpose(0, 1, 4, 2, 5, 3).reshape(R1 * N1 * N2 * R2))
    out3 = _tt_gather(idx3, tab)
    return (out3.reshape(R1, 2, 128, 8, 128)
            .transpose(2, 4, 0, 1, 3).reshape(B, R1, R2))
